# baseline (speedup 1.0000x reference)
import functools

import jax
import jax.numpy as jnp
from jax import lax
from jax.experimental import pallas as pl
from jax.experimental.pallas import tpu as pltpu
from jax.experimental.pallas import tpu_sc as plsc

B, C = 128, 512
R = 32
NCHUNK = C // R
OUT = C * (C + 1) // 2
CC = C * C
SRC_W = R * C
PACKW = 15888 + 16
L = 16

_OFFS = tuple(int(16400 * c - 512 * c * c) for c in range(NCHUNK))
_LENS = tuple(int(15888 - 1024 * c) for c in range(NCHUNK))


def _make_kernel():
    nc, ns = 2, 16
    nw = nc * ns
    bpw = B // nw
    mesh = plsc.VectorSubcoreMesh(core_axis_name="c", subcore_axis_name="s")

    @functools.partial(
        pl.kernel,
        mesh=mesh,
        out_type=jax.ShapeDtypeStruct((B * OUT,), jnp.float32),
        compiler_params=pltpu.CompilerParams(needs_layout_passes=False),
        scratch_types=[
            pltpu.VMEM((2 * R, C), jnp.float32),
            pltpu.VMEM((bpw * PACKW,), jnp.float32),
            pltpu.SemaphoreType.DMA((2,)),
            pltpu.SemaphoreType.DMA((bpw,)),
        ],
    )
    def tri_kernel(x_hbm, out_hbm, src_v, pack_v, s_src, s_out):
        wid = lax.axis_index("s") * nc + lax.axis_index("c")
        b0 = wid * bpw

        def src_desc(i0, t, p):
            voff = pl.multiple_of(p * R, 8)
            return pltpu.make_async_copy(
                x_hbm.at[b0 + t, pl.ds(i0, R), :],
                src_v.at[pl.ds(voff, R), :],
                s_src.at[p])

        def out_desc(c, t):
            ooff = pl.multiple_of((b0 + t) * OUT + _OFFS[c], 8)
            poff = pl.multiple_of(t * PACKW, 8)
            return pltpu.make_async_copy(
                pack_v.at[pl.ds(poff, _LENS[c])],
                out_hbm.at[pl.ds(ooff, _LENS[c])],
                s_out.at[t])

        src_desc(0, 0, 0).start()

        for c in range(NCHUNK):
            i0 = R * c

            def chunk_body(t, _, c=c, i0=i0):
                p = lax.rem(t, 2)

                @pl.when(t < bpw - 1)
                def _():
                    src_desc(i0, t + 1, lax.rem(t + 1, 2)).start()

                if c + 1 < NCHUNK:
                    @pl.when(t == bpw - 1)
                    def _():
                        src_desc(i0 + R, 0, 0).start()

                if c > 0:
                    out_desc(c - 1, t).wait()
                src_desc(i0, t, p).wait()

                dbase = t * PACKW
                rbase = p * R

                def row_body(k, d):
                    rr = (R - 1) - k
                    head = lax.rem(rr, L)
                    cb = (i0 + rr) - head
                    nblk = lax.shift_right_logical(C - cb, 4)
                    row = rbase + rr
                    dst0 = d - head

                    @plsc.parallel_loop(0, nblk, 1, unroll=4)
                    def _(u):
                        pack_v[pl.ds(dst0 + L * u, L)] = \
                            src_v[row, pl.ds(cb + L * u, L)]
                    return d - ((C - i0) - rr + 1)

                d31 = dbase + (R - 1) * (C - i0) - 465
                lax.fori_loop(0, R, row_body, d31)

                out_desc(c, t).start()
                return ()

            lax.fori_loop(0, bpw, chunk_body, ())

        def drain_body(t, _):
            out_desc(NCHUNK - 1, t).wait()
            return ()

        lax.fori_loop(0, bpw, drain_body, ())

    return tri_kernel


_tri_cache = []


def _impl(x):
    if not _tri_cache:
        _tri_cache.append(_make_kernel())
    return _tri_cache[0](x).reshape(B, OUT)


_jit_cache = {}


def kernel(x):
    try:
        dev = next(iter(x.devices()))
    except Exception:
        try:
            dev = jax.devices()[0]
        except Exception:
            dev = None
    fn = _jit_cache.get(dev)
    if fn is None:
        from jax._src.layout import Format, Layout
        from jax.sharding import SingleDeviceSharding
        if dev is None:
            fn = jax.jit(_impl)
        else:
            fmt = Format(Layout((0, 1), tiling=((128,),)),
                         SingleDeviceSharding(dev))
            fn = jax.jit(_impl, out_shardings=fmt)
        _jit_cache[dev] = fn
    return fn(x)


# speedup vs baseline: 1.2642x; 1.2642x over previous
import functools

import jax
import jax.numpy as jnp
from jax import lax
from jax.experimental import pallas as pl
from jax.experimental.pallas import tpu as pltpu
from jax.experimental.pallas import tpu_sc as plsc

B, C = 128, 512
R = 16
NCHUNK = C // R
OUT = C * (C + 1) // 2
L = 16
NQMAX = 64

_LENS = tuple(8072 - 256 * c for c in range(NCHUNK))
_OFFS = []
_o = 0
for _c in range(NCHUNK):
    _OFFS.append(_o)
    _o += _LENS[_c]
_OFFS.append(_o)
assert _OFFS[-1] == OUT
_P = tuple(o % 128 for o in _OFFS)
_QA = tuple(o // 128 for o in _OFFS)
_NQ = tuple((_P[c] + _LENS[c] - _P[c + 1]) // 128 for c in range(NCHUNK))


def _make_kernel():
    nc, ns = 2, 16
    nw = nc * ns
    bpw = B // nw
    mesh = plsc.VectorSubcoreMesh(core_axis_name="c", subcore_axis_name="s")

    @functools.partial(
        pl.kernel,
        mesh=mesh,
        out_type=jax.ShapeDtypeStruct((B // 8, OUT // 128, 8, 128),
                                      jnp.float32),
        compiler_params=pltpu.CompilerParams(needs_layout_passes=False),
        scratch_types=[
            pltpu.VMEM((2 * R, C), jnp.float32),
            pltpu.VMEM((2 * bpw, NQMAX, 128), jnp.float32),
            pltpu.SemaphoreType.DMA((2,)),
            pltpu.SemaphoreType.DMA((2 * bpw,)),
        ],
    )
    def tri_kernel(x_hbm, out_hbm, src_v, pack_v, s_src, s_out):
        wid = lax.axis_index("s") * nc + lax.axis_index("c")
        b0 = wid * bpw
        g = lax.div(b0, 8)
        br0 = lax.rem(b0, 8)

        def src_desc(i0, t, p):
            voff = pl.multiple_of(p * R, 8)
            return pltpu.make_async_copy(
                x_hbm.at[b0 + t, pl.ds(i0, R), :],
                src_v.at[pl.ds(voff, R), :],
                s_src.at[p])

        def out_desc(c, t):
            slot = 2 * t + (c % 2)
            return pltpu.make_async_copy(
                pack_v.at[slot, pl.ds(0, _NQ[c]), :],
                out_hbm.at[g, pl.ds(_QA[c], _NQ[c]), br0 + t, :],
                s_out.at[slot])

        src_desc(0, 0, 0).start()

        for c in range(NCHUNK):
            i0 = R * c

            def chunk_body(t, _, c=c, i0=i0):
                p = lax.rem(t, 2)

                @pl.when(t < bpw - 1)
                def _():
                    src_desc(i0, t + 1, lax.rem(t + 1, 2)).start()

                if c + 1 < NCHUNK:
                    @pl.when(t == bpw - 1)
                    def _():
                        src_desc(i0 + R, 0, 0).start()

                if c > 1:
                    out_desc(c - 2, t).wait()
                src_desc(i0, t, p).wait()

                slot = 2 * t + (c % 2)

                if _P[c] > 0:
                    prev = 2 * t + ((c - 1) % 2)
                    srow = _NQ[c - 1]
                    for k in range((_P[c] + L - 1) // L):
                        pack_v[slot, 0, pl.ds(L * k, L)] = \
                            pack_v[prev, srow, pl.ds(L * k, L)]

                rbase = p * R
                nblk = 32 - c

                def row_body(k, d):
                    rr = (R - 1) - k
                    row = rbase + rr
                    dst0 = d - rr

                    @plsc.parallel_loop(0, nblk, 1, unroll=4)
                    def _(u):
                        f = dst0 + L * u
                        pack_v[slot, lax.shift_right_logical(f, 7),
                               pl.ds(lax.rem(f, 128), L)] = \
                            src_v[row, pl.ds(i0 + L * u, L)]
                    return d - ((C - i0) - rr + 1)

                d15 = _P[c] + (R - 1) * (C - i0) - 105
                lax.fori_loop(0, R, row_body, d15)

                out_desc(c, t).start()
                return ()

            lax.fori_loop(0, bpw, chunk_body, ())

        def drain_body(t, _):
            out_desc(NCHUNK - 2, t).wait()
            out_desc(NCHUNK - 1, t).wait()
            return ()

        lax.fori_loop(0, bpw, drain_body, ())

    return tri_kernel


_tri_cache = []


def kernel(x):
    if not _tri_cache:
        _tri_cache.append(jax.jit(_run))
    return _tri_cache[0](x)


def _run(x):
    if len(_tri_cache) < 2:
        _tri_cache.append(_make_kernel())
    y = _tri_cache[1](x)
    return y.transpose(0, 2, 1, 3).reshape(B, OUT)


# speedup vs baseline: 1.4201x; 1.1233x over previous
import functools

import jax
import jax.numpy as jnp
from jax import lax
from jax.experimental import pallas as pl
from jax.experimental.pallas import tpu as pltpu
from jax.experimental.pallas import tpu_sc as plsc

B, C = 128, 512
R = 16
NCHUNK = C // R
OUT = C * (C + 1) // 2
L = 16
NQMAX = 64

_LENS = tuple(8072 - 256 * c for c in range(NCHUNK))
_OFFS = []
_o = 0
for _c in range(NCHUNK):
    _OFFS.append(_o)
    _o += _LENS[_c]
_OFFS.append(_o)
assert _OFFS[-1] == OUT
_P = tuple(o % 128 for o in _OFFS)
_QA = tuple(o // 128 for o in _OFFS)
_NQ = tuple((_P[c] + _LENS[c] - _P[c + 1]) // 128 for c in range(NCHUNK))


def _make_kernel():
    nc, ns = 2, 16
    nw = nc * ns
    bpw = B // nw
    mesh = plsc.VectorSubcoreMesh(core_axis_name="c", subcore_axis_name="s")

    @functools.partial(
        pl.kernel,
        mesh=mesh,
        out_type=jax.ShapeDtypeStruct((B // 8, OUT // 128, 8, 128),
                                      jnp.float32),
        compiler_params=pltpu.CompilerParams(needs_layout_passes=False),
        scratch_types=[
            pltpu.VMEM((2 * R, C), jnp.float32),
            pltpu.VMEM((2 * bpw, NQMAX, 128), jnp.float32),
            pltpu.SemaphoreType.DMA((2,)),
            pltpu.SemaphoreType.DMA((2 * bpw,)),
        ],
    )
    def tri_kernel(x_hbm, out_hbm, src_v, pack_v, s_src, s_out):
        wid = lax.axis_index("s") * nc + lax.axis_index("c")
        b0 = wid * bpw
        g = lax.div(b0, 8)
        br0 = lax.rem(b0, 8)

        def src_desc(i0, t, p):
            voff = pl.multiple_of(p * R, 8)
            return pltpu.make_async_copy(
                x_hbm.at[b0 + t, pl.ds(i0, R), :],
                src_v.at[pl.ds(voff, R), :],
                s_src.at[p])

        def out_desc(c, t):
            slot = 2 * t + (c % 2)
            return pltpu.make_async_copy(
                pack_v.at[slot, pl.ds(0, _NQ[c]), :],
                out_hbm.at[g, pl.ds(_QA[c], _NQ[c]), br0 + t, :],
                s_out.at[slot])

        src_desc(0, 0, 0).start()

        for c in range(NCHUNK):
            i0 = R * c

            def chunk_body(t, _, c=c, i0=i0):
                p = lax.rem(t, 2)

                @pl.when(t < bpw - 1)
                def _():
                    src_desc(i0, t + 1, lax.rem(t + 1, 2)).start()

                if c + 1 < NCHUNK:
                    @pl.when(t == bpw - 1)
                    def _():
                        src_desc(i0 + R, 0, 0).start()

                if c > 1:
                    out_desc(c - 2, t).wait()
                src_desc(i0, t, p).wait()

                slot = 2 * t + (c % 2)

                if _P[c] > 0:
                    prev = 2 * t + ((c - 1) % 2)
                    srow = _NQ[c - 1]
                    for k in range((_P[c] + L - 1) // L):
                        pack_v[slot, 0, pl.ds(L * k, L)] = \
                            pack_v[prev, srow, pl.ds(L * k, L)]

                rbase = p * R
                nblk = 32 - c

                def row_body(k, d):
                    rr = (R - 1) - k
                    row = rbase + rr
                    dst0 = d - rr

                    @plsc.parallel_loop(0, nblk, 1, unroll=8)
                    def _(u):
                        f = dst0 + L * u
                        pack_v[slot, lax.shift_right_logical(f, 7),
                               pl.ds(lax.rem(f, 128), L)] = \
                            src_v[row, pl.ds(i0 + L * u, L)]
                    return d - ((C - i0) - rr + 1)

                d15 = _P[c] + (R - 1) * (C - i0) - 105
                lax.fori_loop(0, R, row_body, d15, unroll=2)

                out_desc(c, t).start()
                return ()

            lax.fori_loop(0, bpw, chunk_body, ())

        def drain_body(t, _):
            out_desc(NCHUNK - 2, t).wait()
            out_desc(NCHUNK - 1, t).wait()
            return ()

        lax.fori_loop(0, bpw, drain_body, ())

    return tri_kernel


_tri_cache = []


def kernel(x):
    if not _tri_cache:
        _tri_cache.append(jax.jit(_run))
    return _tri_cache[0](x)


def _run(x):
    if len(_tri_cache) < 2:
        _tri_cache.append(_make_kernel())
    y = _tri_cache[1](x)
    return y.transpose(0, 2, 1, 3).reshape(B, OUT)
